# baseline (speedup 1.0000x reference)
import functools

import jax
import jax.numpy as jnp
from jax import lax
from jax.experimental import pallas as pl
from jax.experimental.pallas import tpu as pltpu
from jax.experimental.pallas import tpu_sc as plsc

VOCAB = 1000000
DIM = 64
SEQ = 200
BATCH = 4096

NC = 2
NS = 16
NW = NC * NS
ROWS = BATCH * SEQ
RPW = ROWS // NW
CH = 128
NCH = RPW // CH
LANES = 16


def _sc_body(x2d, tab, pos, out, idx_all, pos2, buf, gsem):
    wid = lax.axis_index("s") * NC + lax.axis_index("c")

    pltpu.sync_copy(x2d.at[pl.ds(wid * NCH, NCH)], idx_all)
    pltpu.sync_copy(pos, pos2.at[pl.ds(0, SEQ)])
    pltpu.sync_copy(pos, pos2.at[pl.ds(SEQ, SEQ)])

    def chunk_body(c, carry):
        pltpu.async_copy(tab.at[idx_all.at[c]], buf, gsem).wait()

        p0 = lax.rem(c * CH, SEQ)

        def jbody(j, carry2):
            pj = p0 + j
            for d in range(DIM // LANES):
                s = pl.ds(d * LANES, LANES)
                buf[j, s] = buf[j, s] + pos2[pj, s]
            return carry2

        lax.fori_loop(0, CH, jbody, 0, unroll=2)

        pltpu.sync_copy(buf, out.at[pl.ds(wid * RPW + c * CH, CH)])
        return carry

    lax.fori_loop(0, NCH, chunk_body, 0)


@functools.partial(jax.jit, static_argnames=())
def _sc_call(x2d, token_table, pos_table):
    mesh = plsc.VectorSubcoreMesh(core_axis_name="c", subcore_axis_name="s")
    return pl.kernel(
        _sc_body,
        out_type=jax.ShapeDtypeStruct((ROWS, DIM), jnp.float32),
        mesh=mesh,
        compiler_params=pltpu.CompilerParams(use_tc_tiling_on_sc=False),
        scratch_types=[
            pltpu.VMEM((NCH, CH), jnp.int32),
            pltpu.VMEM((2 * SEQ, DIM), jnp.float32),
            pltpu.VMEM((CH, DIM), jnp.float32),
            pltpu.SemaphoreType.DMA,
        ],
    )(x2d, token_table, pos_table)


def kernel(x, token_table, pos_table):
    x2d = x.astype(jnp.int32).reshape(ROWS // CH, CH)
    out = _sc_call(x2d, token_table, pos_table)
    return out.reshape(BATCH, SEQ, DIM)


# speedup vs baseline: 1.1293x; 1.1293x over previous
import functools

import jax
import jax.numpy as jnp
from jax import lax
from jax.experimental import pallas as pl
from jax.experimental.pallas import tpu as pltpu
from jax.experimental.pallas import tpu_sc as plsc

VOCAB = 1000000
DIM = 64
SEQ = 200
BATCH = 4096

NC = 2
NS = 16
NW = NC * NS
ROWS = BATCH * SEQ
RPW = ROWS // NW
CH = 128
NCH = RPW // CH
NBUF = 4
LANES = 16


def _sc_body(x2d, tab, pos, out, idx_all, pos2, bufs, gsems, osems):
    wid = lax.axis_index("s") * NC + lax.axis_index("c")

    pltpu.sync_copy(x2d.at[pl.ds(wid * NCH, NCH)], idx_all)
    pltpu.sync_copy(pos, pos2.at[pl.ds(0, SEQ)])
    pltpu.sync_copy(pos, pos2.at[pl.ds(SEQ, SEQ)])

    def start_gather(c, b):
        pltpu.async_copy(tab.at[idx_all.at[c]], bufs[b], gsems[b])

    def wait_gather(b):
        pltpu.make_async_copy(tab.at[pl.ds(0, CH)], bufs[b], gsems[b]).wait()

    def start_out(c, b):
        pltpu.async_copy(bufs[b], out.at[pl.ds(wid * RPW + c * CH, CH)],
                         osems[b])

    def wait_out(b):
        pltpu.make_async_copy(bufs[b], out.at[pl.ds(0, CH)], osems[b]).wait()

    for b in range(NBUF - 1):
        start_gather(b, b)

    def group_body(g, carry):
        for b in range(NBUF):
            c = g * NBUF + b
            f = c + NBUF - 1
            bf = (b + NBUF - 1) % NBUF

            @pl.when(jnp.logical_and(f >= NBUF, f < NCH))
            def _():
                wait_out(bf)
                start_gather(f, bf)

            @pl.when(jnp.logical_and(f < NBUF, f < NCH))
            def _():
                start_gather(f, bf)

            wait_gather(b)

            p0 = lax.rem(c * CH, SEQ)
            buf = bufs[b]

            def jbody(j, carry2):
                pj = p0 + j
                for d in range(DIM // LANES):
                    s = pl.ds(d * LANES, LANES)
                    buf[j, s] = buf[j, s] + pos2[pj, s]
                return carry2

            lax.fori_loop(0, CH, jbody, 0, unroll=4)

            start_out(c, b)
        return carry

    lax.fori_loop(0, NCH // NBUF, group_body, 0)

    for b in range(NBUF):
        wait_out(b)


@jax.jit
def _sc_call(x2d, token_table, pos_table):
    mesh = plsc.VectorSubcoreMesh(core_axis_name="c", subcore_axis_name="s")

    def body(x2d_r, tab_r, pos_r, out_r, idx_all, pos2,
             b0, b1, b2, b3, g0, g1, g2, g3, o0, o1, o2, o3):
        _sc_body(x2d_r, tab_r, pos_r, out_r, idx_all, pos2,
                 [b0, b1, b2, b3], [g0, g1, g2, g3], [o0, o1, o2, o3])

    return pl.kernel(
        body,
        out_type=jax.ShapeDtypeStruct((ROWS, DIM), jnp.float32),
        mesh=mesh,
        compiler_params=pltpu.CompilerParams(use_tc_tiling_on_sc=False),
        scratch_types=(
            [pltpu.VMEM((NCH, CH), jnp.int32),
             pltpu.VMEM((2 * SEQ, DIM), jnp.float32)]
            + [pltpu.VMEM((CH, DIM), jnp.float32) for _ in range(NBUF)]
            + [pltpu.SemaphoreType.DMA for _ in range(2 * NBUF)]
        ),
    )(x2d, token_table, pos_table)


def kernel(x, token_table, pos_table):
    x2d = x.astype(jnp.int32).reshape(ROWS // CH, CH)
    out = _sc_call(x2d, token_table, pos_table)
    return out.reshape(BATCH, SEQ, DIM)


# speedup vs baseline: 1.5395x; 1.3633x over previous
import functools

import jax
import jax.numpy as jnp
from jax import lax
from jax.experimental import pallas as pl
from jax.experimental.pallas import tpu as pltpu
from jax.experimental.pallas import tpu_sc as plsc

VOCAB = 1000000
DIM = 64
SEQ = 200
BATCH = 4096

NC = 2
NS = 16
NW = NC * NS
BPW = BATCH // NW
NCH = SEQ
NBUF = 4
LANES = 16


def _sc_body(xt, tab, pos, out, idx_all, pos_v, bufs, gsems, osems):
    wid = lax.axis_index("s") * NC + lax.axis_index("c")
    b0 = wid * BPW

    pltpu.sync_copy(xt.at[:, pl.ds(b0, BPW)], idx_all)
    pltpu.sync_copy(pos, pos_v)

    def start_gather(c, b):
        pltpu.async_copy(tab.at[idx_all.at[c]], bufs[b], gsems[b])

    def wait_gather(b):
        pltpu.make_async_copy(tab.at[pl.ds(0, BPW)], bufs[b], gsems[b]).wait()

    def start_out(c, b):
        pltpu.async_copy(bufs[b], out.at[c, pl.ds(b0, BPW)], osems[b])

    def wait_out(b):
        pltpu.make_async_copy(bufs[b], out.at[0, pl.ds(b0, BPW)],
                              osems[b]).wait()

    for b in range(NBUF - 1):
        start_gather(b, b)

    def group_body(g, carry):
        for b in range(NBUF):
            c = g * NBUF + b
            f = c + NBUF - 1
            bf = (b + NBUF - 1) % NBUF

            @pl.when(jnp.logical_and(f >= NBUF, f < NCH))
            def _():
                wait_out(bf)
                start_gather(f, bf)

            @pl.when(jnp.logical_and(f < NBUF, f < NCH))
            def _():
                start_gather(f, bf)

            wait_gather(b)

            buf = bufs[b]
            prow = [pos_v[c, pl.ds(d * LANES, LANES)]
                    for d in range(DIM // LANES)]

            def jbody(j, carry2):
                for d in range(DIM // LANES):
                    s = pl.ds(d * LANES, LANES)
                    buf[j, s] = buf[j, s] + prow[d]
                return carry2

            lax.fori_loop(0, BPW, jbody, 0, unroll=4)

            start_out(c, b)
        return carry

    lax.fori_loop(0, NCH // NBUF, group_body, 0)

    for b in range(NBUF):
        wait_out(b)


@jax.jit
def _sc_call(xt, token_table, pos_table):
    mesh = plsc.VectorSubcoreMesh(core_axis_name="c", subcore_axis_name="s")

    def body(xt_r, tab_r, pos_r, out_r, idx_all, pos_v,
             b0, b1, b2, b3, g0, g1, g2, g3, o0, o1, o2, o3):
        _sc_body(xt_r, tab_r, pos_r, out_r, idx_all, pos_v,
                 [b0, b1, b2, b3], [g0, g1, g2, g3], [o0, o1, o2, o3])

    return pl.kernel(
        body,
        out_type=jax.ShapeDtypeStruct((SEQ, BATCH, DIM), jnp.float32),
        mesh=mesh,
        compiler_params=pltpu.CompilerParams(use_tc_tiling_on_sc=False),
        scratch_types=(
            [pltpu.VMEM((SEQ, BPW), jnp.int32),
             pltpu.VMEM((SEQ, DIM), jnp.float32)]
            + [pltpu.VMEM((BPW, DIM), jnp.float32) for _ in range(NBUF)]
            + [pltpu.SemaphoreType.DMA for _ in range(2 * NBUF)]
        ),
    )(xt, token_table, pos_table)


def kernel(x, token_table, pos_table):
    xt = x.astype(jnp.int32).T
    out_t = _sc_call(xt, token_table, pos_table)
    return out_t.transpose(1, 0, 2)
